# baseline (speedup 1.0000x reference)
import jax
import jax.numpy as jnp
from jax.experimental import pallas as pl

_T = 0.05
_BK = 2048


def _matmul_kernel(x_ref, m_ref, o_ref):
    o_ref[...] = jnp.dot(
        x_ref[...], m_ref[...].T, preferred_element_type=jnp.float32
    ) * (1.0 / _T)


def kernel(x, mem_feat):
    q, d = x.shape
    k = mem_feat.shape[0]
    grid = (k // _BK,)
    return pl.pallas_call(
        _matmul_kernel,
        grid=grid,
        in_specs=[
            pl.BlockSpec((q, d), lambda i: (0, 0)),
            pl.BlockSpec((_BK, d), lambda i: (i, 0)),
        ],
        out_specs=pl.BlockSpec((q, _BK), lambda i: (0, i)),
        out_shape=jax.ShapeDtypeStruct((q, k), jnp.float32),
    )(x, mem_feat)


# speedup vs baseline: 1.0278x; 1.0278x over previous
import jax
import jax.numpy as jnp
from jax.experimental import pallas as pl

_T = 0.05
_BK = 4096


def _matmul_kernel(x_ref, m_ref, o_ref):
    o_ref[...] = jnp.dot(
        x_ref[...], m_ref[...].T, preferred_element_type=jnp.float32
    ) * (1.0 / _T)


def kernel(x, mem_feat):
    q, d = x.shape
    k = mem_feat.shape[0]
    grid = (k // _BK,)
    return pl.pallas_call(
        _matmul_kernel,
        grid=grid,
        in_specs=[
            pl.BlockSpec((q, d), lambda i: (0, 0)),
            pl.BlockSpec((_BK, d), lambda i: (i, 0)),
        ],
        out_specs=pl.BlockSpec((q, _BK), lambda i: (0, i)),
        out_shape=jax.ShapeDtypeStruct((q, k), jnp.float32),
    )(x, mem_feat)


# speedup vs baseline: 1.0328x; 1.0049x over previous
import jax
import jax.numpy as jnp
from jax.experimental import pallas as pl

_T = 0.05
_BK = 4096


def _matmul_kernel(x_ref, m_ref, o_ref):
    xb = x_ref[...].astype(jnp.bfloat16)
    mb = m_ref[...].astype(jnp.bfloat16)
    o_ref[...] = jnp.dot(
        xb, mb.T, preferred_element_type=jnp.float32
    ) * (1.0 / _T)


def kernel(x, mem_feat):
    q, d = x.shape
    k = mem_feat.shape[0]
    grid = (k // _BK,)
    return pl.pallas_call(
        _matmul_kernel,
        grid=grid,
        in_specs=[
            pl.BlockSpec((q, d), lambda i: (0, 0)),
            pl.BlockSpec((_BK, d), lambda i: (i, 0)),
        ],
        out_specs=pl.BlockSpec((q, _BK), lambda i: (0, i)),
        out_shape=jax.ShapeDtypeStruct((q, k), jnp.float32),
    )(x, mem_feat)
